# baseline (speedup 1.0000x reference)
import functools

import jax
import jax.numpy as jnp
import numpy as np
from jax import lax
from jax.experimental import pallas as pl
from jax.experimental.pallas import tpu as pltpu
from jax.experimental.pallas import tpu_sc as plsc

B = 8
N = 8192
S = 2048
D = 256
NC = 2
NS = 16
NW = NC * NS
WPB = NW // B
ROWS_PER_W = (B * S) // NW
CH = 128
NCH = ROWS_PER_W // CH
L = 16


def _rotl32(x, d):
    d = np.uint32(d)
    return (x << d) | (x >> np.uint32(32 - d))


def _threefry2x32_np(k1, k2, x0, x1):
    k1 = np.uint32(k1)
    k2 = np.uint32(k2)
    x0 = x0.astype(np.uint32).copy()
    x1 = x1.astype(np.uint32).copy()
    ks = [k1, k2, k1 ^ k2 ^ np.uint32(0x1BD11BDA)]
    rotations = [(13, 15, 26, 6), (17, 29, 16, 24)]
    x0 = x0 + ks[0]
    x1 = x1 + ks[1]
    for i in range(5):
        for r in rotations[i % 2]:
            x0 = x0 + x1
            x1 = _rotl32(x1, r)
            x1 = x0 ^ x1
        x0 = x0 + ks[(i + 1) % 3]
        x1 = x1 + ks[(i + 2) % 3] + np.uint32(i + 1)
    return x0, x1


def _np_permutation(seed, n):
    key = (np.uint32(0), np.uint32(seed))
    x = np.arange(n, dtype=np.int32)
    num_rounds = int(np.ceil(3 * np.log(max(1, n)) / np.log(2**32 - 1)))
    for _ in range(num_rounds):
        b1, b2 = _threefry2x32_np(
            key[0], key[1], np.zeros(2, np.uint32), np.arange(2, dtype=np.uint32)
        )
        key, subkey = (b1[0], b2[0]), (b1[1], b2[1])
        b1, b2 = _threefry2x32_np(
            subkey[0], subkey[1], np.zeros(n, np.uint32),
            np.arange(n, dtype=np.uint32),
        )
        x = x[np.argsort(b1 ^ b2, kind="stable")]
    return x


_CHOICE = _np_permutation(42, N)[:S].astype(np.int32)
_IDX_OUT = np.tile(_CHOICE[None, :], (B, 1))
_WIDX = _CHOICE.reshape(WPB, NCH, CH)


def _sc_gather(pos, x, widx):
    mesh = plsc.VectorSubcoreMesh(core_axis_name="c", subcore_axis_name="s")

    @functools.partial(
        pl.kernel,
        mesh=mesh,
        compiler_params=pltpu.CompilerParams(needs_layout_passes=False),
        out_type=[
            jax.ShapeDtypeStruct((B, S, D), jnp.float32),
            jax.ShapeDtypeStruct((B * S * 3,), jnp.float32),
        ],
        scratch_types=[
            pltpu.VMEM((NCH, CH), jnp.int32),
            pltpu.VMEM((2, CH, D), jnp.float32),
            pltpu.VMEM((N * 3,), jnp.float32),
            pltpu.VMEM((ROWS_PER_W * 3,), jnp.float32),
            pltpu.SemaphoreType.DMA,
            pltpu.SemaphoreType.DMA,
            pltpu.SemaphoreType.DMA,
        ],
    )
    def k(pf, xf, gi, xout, pout, idx_v, xbuf, ptbl, pbuf, sem0, sem1, psem):
        wid = lax.axis_index("s") * NC + lax.axis_index("c")
        b = wid // WPB
        q = wid % WPB
        pltpu.sync_copy(gi.at[q], idx_v)

        pos_cp = pltpu.async_copy(pf.at[pl.ds(b * N * 3, N * 3)], ptbl, psem)

        sems = (sem0, sem1)
        xb = xf.at[b]
        xo = xout.at[b]
        cps = [
            pltpu.async_copy(xb.at[idx_v.at[0]], xbuf.at[0], sems[0]),
            pltpu.async_copy(xb.at[idx_v.at[1]], xbuf.at[1], sems[1]),
        ]
        for c in range(NCH):
            p = c % 2
            cps[p].wait()
            pltpu.sync_copy(
                xbuf.at[p], xo.at[pl.ds(q * ROWS_PER_W + c * CH, CH)]
            )
            if c + 2 < NCH:
                cps[p] = pltpu.async_copy(
                    xb.at[idx_v.at[c + 2]], xbuf.at[p], sems[p]
                )

        pos_cp.wait()
        iota = lax.iota(jnp.int32, L)
        for v in range(ROWS_PER_W // L):
            rows = idx_v[v // (CH // L), pl.ds((v % (CH // L)) * L, L)]
            for col in range(3):
                vals = plsc.load_gather(ptbl, [rows * 3 + col])
                plsc.store_scatter(pbuf, [iota * 3 + (v * 3 * L + col)], vals)
        base = wid * ROWS_PER_W
        pltpu.sync_copy(pbuf, pout.at[pl.ds(base * 3, ROWS_PER_W * 3)])

    return k(pos, x, widx)


def kernel(pos, x):
    posflat = pos.reshape(B * N * 3)
    xo, po = _sc_gather(posflat, x, jnp.asarray(_WIDX))
    idx = jnp.asarray(_IDX_OUT)
    return (idx, po.reshape(B, S, 3), xo)


# speedup vs baseline: 1.3592x; 1.3592x over previous
import functools

import jax
import jax.numpy as jnp
import numpy as np
from jax import lax
from jax.experimental import pallas as pl
from jax.experimental.pallas import tpu as pltpu
from jax.experimental.pallas import tpu_sc as plsc

B = 8
N = 8192
S = 2048
D = 256
NC = 2
NS = 16
NW = NC * NS
WPB = NW // B
ROWS_PER_W = (B * S) // NW
CH = 128
NCH = ROWS_PER_W // CH
L = 16


def _rotl32(x, d):
    d = np.uint32(d)
    return (x << d) | (x >> np.uint32(32 - d))


def _threefry2x32_np(k1, k2, x0, x1):
    k1 = np.uint32(k1)
    k2 = np.uint32(k2)
    x0 = x0.astype(np.uint32).copy()
    x1 = x1.astype(np.uint32).copy()
    ks = [k1, k2, k1 ^ k2 ^ np.uint32(0x1BD11BDA)]
    rotations = [(13, 15, 26, 6), (17, 29, 16, 24)]
    x0 = x0 + ks[0]
    x1 = x1 + ks[1]
    for i in range(5):
        for r in rotations[i % 2]:
            x0 = x0 + x1
            x1 = _rotl32(x1, r)
            x1 = x0 ^ x1
        x0 = x0 + ks[(i + 1) % 3]
        x1 = x1 + ks[(i + 2) % 3] + np.uint32(i + 1)
    return x0, x1


def _np_permutation(seed, n):
    key = (np.uint32(0), np.uint32(seed))
    x = np.arange(n, dtype=np.int32)
    num_rounds = int(np.ceil(3 * np.log(max(1, n)) / np.log(2**32 - 1)))
    for _ in range(num_rounds):
        b1, b2 = _threefry2x32_np(
            key[0], key[1], np.zeros(2, np.uint32), np.arange(2, dtype=np.uint32)
        )
        key, subkey = (b1[0], b2[0]), (b1[1], b2[1])
        b1, b2 = _threefry2x32_np(
            subkey[0], subkey[1], np.zeros(n, np.uint32),
            np.arange(n, dtype=np.uint32),
        )
        x = x[np.argsort(b1 ^ b2, kind="stable")]
    return x


_CHOICE = _np_permutation(42, N)[:S].astype(np.int32)
_IDX_OUT = np.tile(_CHOICE[None, :], (B, 1))
_WIDX = _CHOICE.reshape(WPB, NCH, CH)


def _sc_gather(pos2d, x, widx):
    mesh = plsc.VectorSubcoreMesh(core_axis_name="c", subcore_axis_name="s")

    @functools.partial(
        pl.kernel,
        mesh=mesh,
        compiler_params=pltpu.CompilerParams(needs_layout_passes=False),
        out_type=[
            jax.ShapeDtypeStruct((B, S, D), jnp.float32),
            jax.ShapeDtypeStruct((B * S, 3), jnp.float32),
        ],
        scratch_types=[
            pltpu.VMEM((NCH, CH), jnp.int32),
            pltpu.VMEM((2, CH, D), jnp.float32),
            pltpu.VMEM((2, CH, 3), jnp.float32),
            pltpu.SemaphoreType.DMA,
            pltpu.SemaphoreType.DMA,
            pltpu.SemaphoreType.DMA,
            pltpu.SemaphoreType.DMA,
        ],
    )
    def k(pf, xf, gi, xout, pout, idx_v, xbuf, pbuf,
          sem0, sem1, psem0, psem1):
        wid = lax.axis_index("s") * NC + lax.axis_index("c")
        b = wid // WPB
        q = wid % WPB
        pltpu.sync_copy(gi.at[q], idx_v)

        sems = (sem0, sem1)
        psems = (psem0, psem1)
        xb = xf.at[b]
        xo = xout.at[b]
        cps = [
            pltpu.async_copy(xb.at[idx_v.at[0]], xbuf.at[0], sems[0]),
            pltpu.async_copy(xb.at[idx_v.at[1]], xbuf.at[1], sems[1]),
        ]

        row0 = b * N

        def pos_fire_chunk(c, p):
            hs = []
            for g in range(CH // L):
                rows = idx_v[c, pl.ds(g * L, L)]
                for j in range(L):
                    row = rows[j] + row0
                    hs.append(
                        pltpu.async_copy(
                            pf.at[pl.ds(row, 1)],
                            pbuf.at[p].at[pl.ds(g * L + j, 1)],
                            psems[p],
                        )
                    )
            return hs

        phs = [pos_fire_chunk(0, 0), pos_fire_chunk(1, 1)]
        pbase = wid * ROWS_PER_W
        for c in range(NCH):
            p = c % 2
            for h in phs[p]:
                h.wait()
            pltpu.sync_copy(
                pbuf.at[p], pout.at[pl.ds(pbase + c * CH, CH)]
            )
            if c + 2 < NCH:
                phs[p] = pos_fire_chunk(c + 2, p)

        for c in range(NCH):
            p = c % 2
            cps[p].wait()
            pltpu.sync_copy(
                xbuf.at[p], xo.at[pl.ds(q * ROWS_PER_W + c * CH, CH)]
            )
            if c + 2 < NCH:
                cps[p] = pltpu.async_copy(
                    xb.at[idx_v.at[c + 2]], xbuf.at[p], sems[p]
                )

    return k(pos2d, x, widx)


def kernel(pos, x):
    pos2d = pos.reshape(B * N, 3)
    xo, po = _sc_gather(pos2d, x, jnp.asarray(_WIDX))
    idx = jnp.asarray(_IDX_OUT)
    return (idx, po.reshape(B, S, 3), xo)


# speedup vs baseline: 1.4355x; 1.0562x over previous
import functools

import jax
import jax.numpy as jnp
import numpy as np
from jax import lax
from jax.experimental import pallas as pl
from jax.experimental.pallas import tpu as pltpu
from jax.experimental.pallas import tpu_sc as plsc

B = 8
N = 8192
S = 2048
D = 256
NC = 2
NS = 16
NW = NC * NS
WPB = NW // B
ROWS_PER_W = (B * S) // NW
CH = 128
NCH = ROWS_PER_W // CH
L = 16


def _rotl32(x, d):
    d = np.uint32(d)
    return (x << d) | (x >> np.uint32(32 - d))


def _threefry2x32_np(k1, k2, x0, x1):
    k1 = np.uint32(k1)
    k2 = np.uint32(k2)
    x0 = x0.astype(np.uint32).copy()
    x1 = x1.astype(np.uint32).copy()
    ks = [k1, k2, k1 ^ k2 ^ np.uint32(0x1BD11BDA)]
    rotations = [(13, 15, 26, 6), (17, 29, 16, 24)]
    x0 = x0 + ks[0]
    x1 = x1 + ks[1]
    for i in range(5):
        for r in rotations[i % 2]:
            x0 = x0 + x1
            x1 = _rotl32(x1, r)
            x1 = x0 ^ x1
        x0 = x0 + ks[(i + 1) % 3]
        x1 = x1 + ks[(i + 2) % 3] + np.uint32(i + 1)
    return x0, x1


def _np_permutation(seed, n):
    key = (np.uint32(0), np.uint32(seed))
    x = np.arange(n, dtype=np.int32)
    num_rounds = int(np.ceil(3 * np.log(max(1, n)) / np.log(2**32 - 1)))
    for _ in range(num_rounds):
        b1, b2 = _threefry2x32_np(
            key[0], key[1], np.zeros(2, np.uint32), np.arange(2, dtype=np.uint32)
        )
        key, subkey = (b1[0], b2[0]), (b1[1], b2[1])
        b1, b2 = _threefry2x32_np(
            subkey[0], subkey[1], np.zeros(n, np.uint32),
            np.arange(n, dtype=np.uint32),
        )
        x = x[np.argsort(b1 ^ b2, kind="stable")]
    return x


_CHOICE = _np_permutation(42, N)[:S].astype(np.int32)
_IDX_OUT = np.tile(_CHOICE[None, :], (B, 1))
_WIDX = _CHOICE.reshape(WPB, NCH, CH)


def _sc_gather(pos2d, x, widx):
    mesh = plsc.VectorSubcoreMesh(core_axis_name="c", subcore_axis_name="s")

    @functools.partial(
        pl.kernel,
        mesh=mesh,
        compiler_params=pltpu.CompilerParams(needs_layout_passes=False),
        out_type=[
            jax.ShapeDtypeStruct((B, S, D), jnp.float32),
            jax.ShapeDtypeStruct((B, S, 3), jnp.float32),
        ],
        scratch_types=[
            pltpu.VMEM((NCH, CH), jnp.int32),
            pltpu.VMEM((2, CH, D), jnp.float32),
            pltpu.VMEM((2, CH, 3), jnp.float32),
            pltpu.SemaphoreType.DMA,
            pltpu.SemaphoreType.DMA,
            pltpu.SemaphoreType.DMA,
            pltpu.SemaphoreType.DMA,
        ],
    )
    def k(pf, xf, gi, xout, pout, idx_v, xbuf, pbuf,
          sem0, sem1, psem0, psem1):
        wid = lax.axis_index("s") * NC + lax.axis_index("c")
        b = wid // WPB
        q = wid % WPB
        pltpu.sync_copy(gi.at[q], idx_v)

        sems = (sem0, sem1)
        psems = (psem0, psem1)
        xb = xf.at[b]
        xo = xout.at[b]
        cps = [
            pltpu.async_copy(xb.at[idx_v.at[0]], xbuf.at[0], sems[0]),
            pltpu.async_copy(xb.at[idx_v.at[1]], xbuf.at[1], sems[1]),
        ]

        pb = pf.at[b]
        po = pout.at[b]

        def pos_fire_chunk(c, p):
            hs = []
            for g in range(CH // L):
                rows = idx_v[c, pl.ds(g * L, L)]
                for j in range(L):
                    row = rows[j]
                    hs.append(
                        pltpu.async_copy(
                            pb.at[pl.ds(row, 1)],
                            pbuf.at[p].at[pl.ds(g * L + j, 1)],
                            psems[p],
                        )
                    )
            return hs

        phs = [pos_fire_chunk(0, 0), pos_fire_chunk(1, 1)]
        for c in range(NCH):
            p = c % 2
            for h in phs[p]:
                h.wait()
            pltpu.sync_copy(
                pbuf.at[p], po.at[pl.ds(q * ROWS_PER_W + c * CH, CH)]
            )
            if c + 2 < NCH:
                phs[p] = pos_fire_chunk(c + 2, p)

        for c in range(NCH):
            p = c % 2
            cps[p].wait()
            pltpu.sync_copy(
                xbuf.at[p], xo.at[pl.ds(q * ROWS_PER_W + c * CH, CH)]
            )
            if c + 2 < NCH:
                cps[p] = pltpu.async_copy(
                    xb.at[idx_v.at[c + 2]], xbuf.at[p], sems[p]
                )

    return k(pos2d, x, widx)


def kernel(pos, x):
    xo, po = _sc_gather(pos, x, jnp.asarray(_WIDX))
    idx = jnp.asarray(_IDX_OUT)
    return (idx, po, xo)


# speedup vs baseline: 2.4802x; 1.7277x over previous
import functools

import jax
import jax.numpy as jnp
import numpy as np
from jax import lax
from jax.experimental import pallas as pl
from jax.experimental.pallas import tpu as pltpu
from jax.experimental.pallas import tpu_sc as plsc

B = 8
N = 8192
S = 2048
D = 256
NC = 2
NS = 16
NW = NC * NS
WPB = NW // B
ROWS_PER_W = (B * S) // NW
CH = 128
NCH = ROWS_PER_W // CH
L = 16


def _rotl32(x, d):
    d = np.uint32(d)
    return (x << d) | (x >> np.uint32(32 - d))


def _threefry2x32_np(k1, k2, x0, x1):
    k1 = np.uint32(k1)
    k2 = np.uint32(k2)
    x0 = x0.astype(np.uint32).copy()
    x1 = x1.astype(np.uint32).copy()
    ks = [k1, k2, k1 ^ k2 ^ np.uint32(0x1BD11BDA)]
    rotations = [(13, 15, 26, 6), (17, 29, 16, 24)]
    x0 = x0 + ks[0]
    x1 = x1 + ks[1]
    for i in range(5):
        for r in rotations[i % 2]:
            x0 = x0 + x1
            x1 = _rotl32(x1, r)
            x1 = x0 ^ x1
        x0 = x0 + ks[(i + 1) % 3]
        x1 = x1 + ks[(i + 2) % 3] + np.uint32(i + 1)
    return x0, x1


def _np_permutation(seed, n):
    key = (np.uint32(0), np.uint32(seed))
    x = np.arange(n, dtype=np.int32)
    num_rounds = int(np.ceil(3 * np.log(max(1, n)) / np.log(2**32 - 1)))
    for _ in range(num_rounds):
        b1, b2 = _threefry2x32_np(
            key[0], key[1], np.zeros(2, np.uint32), np.arange(2, dtype=np.uint32)
        )
        key, subkey = (b1[0], b2[0]), (b1[1], b2[1])
        b1, b2 = _threefry2x32_np(
            subkey[0], subkey[1], np.zeros(n, np.uint32),
            np.arange(n, dtype=np.uint32),
        )
        x = x[np.argsort(b1 ^ b2, kind="stable")]
    return x


_CHOICE = _np_permutation(42, N)[:S].astype(np.int32)
_IDX_OUT = np.tile(_CHOICE[None, :], (B, 1))
_WIDX = _CHOICE.reshape(WPB, NCH, CH)


def _sc_gather(pos_t, x, widx):
    mesh = plsc.VectorSubcoreMesh(core_axis_name="c", subcore_axis_name="s")

    @functools.partial(
        pl.kernel,
        mesh=mesh,
        compiler_params=pltpu.CompilerParams(needs_layout_passes=False),
        out_type=[
            jax.ShapeDtypeStruct((B, S, D), jnp.float32),
            jax.ShapeDtypeStruct((3, B, S), jnp.float32),
        ],
        scratch_types=[
            pltpu.VMEM((NCH, CH), jnp.int32),
            pltpu.VMEM((2, CH, D), jnp.float32),
            pltpu.VMEM((3 * N,), jnp.float32),
            pltpu.VMEM((3 * ROWS_PER_W,), jnp.float32),
            pltpu.SemaphoreType.DMA,
            pltpu.SemaphoreType.DMA,
            pltpu.SemaphoreType.DMA,
        ],
    )
    def k(pt, xf, gi, xout, pout, idx_v, xbuf, ptbl, pbuf, sem0, sem1, psem):
        wid = lax.axis_index("s") * NC + lax.axis_index("c")
        b = wid // WPB
        q = wid % WPB
        pltpu.sync_copy(gi.at[q], idx_v)

        pcps = [
            pltpu.async_copy(
                pt.at[c3, b], ptbl.at[pl.ds(c3 * N, N)], psem
            )
            for c3 in range(3)
        ]

        sems = (sem0, sem1)
        xb = xf.at[b]
        xo = xout.at[b]
        cps = [
            pltpu.async_copy(xb.at[idx_v.at[0]], xbuf.at[0], sems[0]),
            pltpu.async_copy(xb.at[idx_v.at[1]], xbuf.at[1], sems[1]),
        ]
        for c in range(NCH):
            p = c % 2
            cps[p].wait()
            pltpu.sync_copy(
                xbuf.at[p], xo.at[pl.ds(q * ROWS_PER_W + c * CH, CH)]
            )
            if c + 2 < NCH:
                cps[p] = pltpu.async_copy(
                    xb.at[idx_v.at[c + 2]], xbuf.at[p], sems[p]
                )

        for cp in pcps:
            cp.wait()
        for v in range(ROWS_PER_W // L):
            rows = idx_v[v // (CH // L), pl.ds((v % (CH // L)) * L, L)]
            for c3 in range(3):
                vals = plsc.load_gather(ptbl, [rows + c3 * N])
                pbuf[pl.ds(c3 * ROWS_PER_W + v * L, L)] = vals
        for c3 in range(3):
            pltpu.sync_copy(
                pbuf.at[pl.ds(c3 * ROWS_PER_W, ROWS_PER_W)],
                pout.at[c3, b].at[pl.ds(q * ROWS_PER_W, ROWS_PER_W)],
            )

    return k(pos_t, x, widx)


def kernel(pos, x):
    pos_t = jnp.transpose(pos, (2, 0, 1))
    xo, po_t = _sc_gather(pos_t, x, jnp.asarray(_WIDX))
    idx = jnp.asarray(_IDX_OUT)
    return (idx, jnp.transpose(po_t, (1, 2, 0)), xo)


# speedup vs baseline: 2.5959x; 1.0466x over previous
import functools

import jax
import jax.numpy as jnp
import numpy as np
from jax import lax
from jax.experimental import pallas as pl
from jax.experimental.pallas import tpu as pltpu
from jax.experimental.pallas import tpu_sc as plsc

B = 8
N = 8192
S = 2048
D = 256
NC = 2
NS = 16
NW = NC * NS
WPB = NW // B
ROWS_PER_W = (B * S) // NW
CH = 128
NCH = ROWS_PER_W // CH
L = 16


def _rotl32(x, d):
    d = np.uint32(d)
    return (x << d) | (x >> np.uint32(32 - d))


def _threefry2x32_np(k1, k2, x0, x1):
    k1 = np.uint32(k1)
    k2 = np.uint32(k2)
    x0 = x0.astype(np.uint32).copy()
    x1 = x1.astype(np.uint32).copy()
    ks = [k1, k2, k1 ^ k2 ^ np.uint32(0x1BD11BDA)]
    rotations = [(13, 15, 26, 6), (17, 29, 16, 24)]
    x0 = x0 + ks[0]
    x1 = x1 + ks[1]
    for i in range(5):
        for r in rotations[i % 2]:
            x0 = x0 + x1
            x1 = _rotl32(x1, r)
            x1 = x0 ^ x1
        x0 = x0 + ks[(i + 1) % 3]
        x1 = x1 + ks[(i + 2) % 3] + np.uint32(i + 1)
    return x0, x1


def _np_permutation(seed, n):
    key = (np.uint32(0), np.uint32(seed))
    x = np.arange(n, dtype=np.int32)
    num_rounds = int(np.ceil(3 * np.log(max(1, n)) / np.log(2**32 - 1)))
    for _ in range(num_rounds):
        b1, b2 = _threefry2x32_np(
            key[0], key[1], np.zeros(2, np.uint32), np.arange(2, dtype=np.uint32)
        )
        key, subkey = (b1[0], b2[0]), (b1[1], b2[1])
        b1, b2 = _threefry2x32_np(
            subkey[0], subkey[1], np.zeros(n, np.uint32),
            np.arange(n, dtype=np.uint32),
        )
        x = x[np.argsort(b1 ^ b2, kind="stable")]
    return x


_CHOICE = _np_permutation(42, N)[:S].astype(np.int32)
_IDX_OUT = np.tile(_CHOICE[None, :], (B, 1))
_WIDX = _CHOICE.reshape(WPB, NCH, CH)


def _sc_gather(pos_t, x, widx):
    mesh = plsc.VectorSubcoreMesh(core_axis_name="c", subcore_axis_name="s")

    @functools.partial(
        pl.kernel,
        mesh=mesh,
        compiler_params=pltpu.CompilerParams(needs_layout_passes=False),
        out_type=[
            jax.ShapeDtypeStruct((B, S, D), jnp.float32),
            jax.ShapeDtypeStruct((3, B, S), jnp.float32),
        ],
        scratch_types=[
            pltpu.VMEM((NCH, CH), jnp.int32),
            pltpu.VMEM((3, CH, D), jnp.float32),
            pltpu.VMEM((3 * N,), jnp.float32),
            pltpu.VMEM((3 * ROWS_PER_W,), jnp.float32),
            pltpu.SemaphoreType.DMA,
            pltpu.SemaphoreType.DMA,
            pltpu.SemaphoreType.DMA,
            pltpu.SemaphoreType.DMA,
            pltpu.SemaphoreType.DMA,
        ],
    )
    def k(pt, xf, gi, xout, pout, idx_v, xbuf, ptbl, pbuf,
          sem0, sem1, sem2, wsem, psem):
        wid = lax.axis_index("s") * NC + lax.axis_index("c")
        b = wid // WPB
        q = wid % WPB
        rbase = q * ROWS_PER_W
        pltpu.sync_copy(gi.at[q], idx_v)

        pcps = [
            pltpu.async_copy(
                pt.at[c3, b], ptbl.at[pl.ds(c3 * N, N)], psem
            )
            for c3 in range(3)
        ]

        gsems = (sem0, sem1, sem2)
        xb = xf.at[b]
        xo = xout.at[b]
        cps = [
            pltpu.async_copy(xb.at[idx_v.at[c]], xbuf.at[c], gsems[c])
            for c in range(3)
        ]

        for cp in pcps:
            cp.wait()
        for v in range(ROWS_PER_W // L):
            rows = idx_v[v // (CH // L), pl.ds((v % (CH // L)) * L, L)]
            for c3 in range(3):
                vals = plsc.load_gather(ptbl, [rows + c3 * N])
                pbuf[pl.ds(c3 * ROWS_PER_W + v * L, L)] = vals
        pws = [
            pltpu.async_copy(
                pbuf.at[pl.ds(c3 * ROWS_PER_W, ROWS_PER_W)],
                pout.at[c3, b].at[pl.ds(rbase, ROWS_PER_W)],
                psem,
            )
            for c3 in range(3)
        ]

        ws = [None] * NCH
        for c in range(NCH):
            p = c % 3
            cps[p].wait()
            ws[c] = pltpu.async_copy(
                xbuf.at[p], xo.at[pl.ds(rbase + c * CH, CH)], wsem
            )
            if c + 3 < NCH:
                ws[c].wait()
                cps[p] = pltpu.async_copy(
                    xb.at[idx_v.at[c + 3]], xbuf.at[p], gsems[p]
                )
        for c in range(NCH):
            if c + 3 >= NCH and ws[c] is not None:
                ws[c].wait()
        for pw in pws:
            pw.wait()

    return k(pos_t, x, widx)


def kernel(pos, x):
    pos_t = jnp.transpose(pos, (2, 0, 1))
    xo, po_t = _sc_gather(pos_t, x, jnp.asarray(_WIDX))
    idx = jnp.asarray(_IDX_OUT)
    return (idx, jnp.transpose(po_t, (1, 2, 0)), xo)


# speedup vs baseline: 2.6183x; 1.0086x over previous
import functools

import jax
import jax.numpy as jnp
import numpy as np
from jax import lax
from jax.experimental import pallas as pl
from jax.experimental.pallas import tpu as pltpu
from jax.experimental.pallas import tpu_sc as plsc

B = 8
N = 8192
S = 2048
D = 256
NC = 2
NS = 16
NW = NC * NS
WPB = NW // B
ROWS_PER_W = (B * S) // NW
CH = 128
NCH = ROWS_PER_W // CH
L = 16


def _rotl32(x, d):
    d = np.uint32(d)
    return (x << d) | (x >> np.uint32(32 - d))


def _threefry2x32_np(k1, k2, x0, x1):
    k1 = np.uint32(k1)
    k2 = np.uint32(k2)
    x0 = x0.astype(np.uint32).copy()
    x1 = x1.astype(np.uint32).copy()
    ks = [k1, k2, k1 ^ k2 ^ np.uint32(0x1BD11BDA)]
    rotations = [(13, 15, 26, 6), (17, 29, 16, 24)]
    x0 = x0 + ks[0]
    x1 = x1 + ks[1]
    for i in range(5):
        for r in rotations[i % 2]:
            x0 = x0 + x1
            x1 = _rotl32(x1, r)
            x1 = x0 ^ x1
        x0 = x0 + ks[(i + 1) % 3]
        x1 = x1 + ks[(i + 2) % 3] + np.uint32(i + 1)
    return x0, x1


def _np_permutation(seed, n):
    key = (np.uint32(0), np.uint32(seed))
    x = np.arange(n, dtype=np.int32)
    num_rounds = int(np.ceil(3 * np.log(max(1, n)) / np.log(2**32 - 1)))
    for _ in range(num_rounds):
        b1, b2 = _threefry2x32_np(
            key[0], key[1], np.zeros(2, np.uint32), np.arange(2, dtype=np.uint32)
        )
        key, subkey = (b1[0], b2[0]), (b1[1], b2[1])
        b1, b2 = _threefry2x32_np(
            subkey[0], subkey[1], np.zeros(n, np.uint32),
            np.arange(n, dtype=np.uint32),
        )
        x = x[np.argsort(b1 ^ b2, kind="stable")]
    return x


_CHOICE = _np_permutation(42, N)[:S].astype(np.int32)
_IDX_OUT = np.tile(_CHOICE[None, :], (B, 1))
_WIDX = _CHOICE.reshape(WPB, NCH, CH)


def _sc_gather(pos_t, x, widx):
    mesh = plsc.VectorSubcoreMesh(core_axis_name="c", subcore_axis_name="s")

    @functools.partial(
        pl.kernel,
        mesh=mesh,
        compiler_params=pltpu.CompilerParams(needs_layout_passes=False),
        out_type=[
            jax.ShapeDtypeStruct((B, S, D), jnp.float32),
            jax.ShapeDtypeStruct((3, B, S), jnp.float32),
        ],
        scratch_types=[
            pltpu.VMEM((NCH, CH), jnp.int32),
            pltpu.VMEM((3, CH, D), jnp.float32),
            pltpu.VMEM((3 * N,), jnp.float32),
            pltpu.VMEM((3 * ROWS_PER_W,), jnp.float32),
            pltpu.SemaphoreType.DMA,
            pltpu.SemaphoreType.DMA,
            pltpu.SemaphoreType.DMA,
            pltpu.SemaphoreType.DMA,
            pltpu.SemaphoreType.DMA,
        ],
    )
    def k(pt, xf, gi, xout, pout, idx_v, xbuf, ptbl, pbuf,
          sem0, sem1, sem2, wsem, psem):
        wid = lax.axis_index("s") * NC + lax.axis_index("c")
        b = wid // WPB
        q = wid % WPB
        rbase = q * ROWS_PER_W
        pltpu.sync_copy(gi.at[q], idx_v)

        pcps = [
            pltpu.async_copy(
                pt.at[c3, b], ptbl.at[pl.ds(c3 * N, N)], psem
            )
            for c3 in range(3)
        ]

        gsems = (sem0, sem1, sem2)
        xb = xf.at[b]
        xo = xout.at[b]
        cps = [
            pltpu.async_copy(xb.at[idx_v.at[c]], xbuf.at[c], gsems[c])
            for c in range(3)
        ]

        NV = ROWS_PER_W // L
        VPC = NV // NCH

        def pos_group(v):
            rows = idx_v[v // (CH // L), pl.ds((v % (CH // L)) * L, L)]
            for c3 in range(3):
                vals = plsc.load_gather(ptbl, [rows + c3 * N])
                pbuf[pl.ds(c3 * ROWS_PER_W + v * L, L)] = vals

        ws = [None] * NCH
        for c in range(NCH):
            p = c % 3
            cps[p].wait()
            ws[c] = pltpu.async_copy(
                xbuf.at[p], xo.at[pl.ds(rbase + c * CH, CH)], wsem
            )
            if c + 3 < NCH:
                ws[c].wait()
                cps[p] = pltpu.async_copy(
                    xb.at[idx_v.at[c + 3]], xbuf.at[p], gsems[p]
                )
            if c == 0:
                for cp in pcps:
                    cp.wait()
            for v in range(c * VPC, (c + 1) * VPC):
                pos_group(v)
        pws = [
            pltpu.async_copy(
                pbuf.at[pl.ds(c3 * ROWS_PER_W, ROWS_PER_W)],
                pout.at[c3, b].at[pl.ds(rbase, ROWS_PER_W)],
                psem,
            )
            for c3 in range(3)
        ]
        for c in range(NCH):
            if c + 3 >= NCH and ws[c] is not None:
                ws[c].wait()
        for pw in pws:
            pw.wait()

    return k(pos_t, x, widx)


def kernel(pos, x):
    pos_t = jnp.transpose(pos, (2, 0, 1))
    xo, po_t = _sc_gather(pos_t, x, jnp.asarray(_WIDX))
    idx = jnp.asarray(_IDX_OUT)
    return (idx, jnp.transpose(po_t, (1, 2, 0)), xo)


# speedup vs baseline: 2.6485x; 1.0115x over previous
import functools

import jax
import jax.numpy as jnp
import numpy as np
from jax import lax
from jax.experimental import pallas as pl
from jax.experimental.pallas import tpu as pltpu
from jax.experimental.pallas import tpu_sc as plsc

B = 8
N = 8192
S = 2048
D = 256
NC = 2
NS = 16
NW = NC * NS
WPB = NW // B
ROWS_PER_W = (B * S) // NW
CH = 128
NCH = ROWS_PER_W // CH
L = 16


def _rotl32(x, d):
    d = np.uint32(d)
    return (x << d) | (x >> np.uint32(32 - d))


def _threefry2x32_np(k1, k2, x0, x1):
    k1 = np.uint32(k1)
    k2 = np.uint32(k2)
    x0 = x0.astype(np.uint32).copy()
    x1 = x1.astype(np.uint32).copy()
    ks = [k1, k2, k1 ^ k2 ^ np.uint32(0x1BD11BDA)]
    rotations = [(13, 15, 26, 6), (17, 29, 16, 24)]
    x0 = x0 + ks[0]
    x1 = x1 + ks[1]
    for i in range(5):
        for r in rotations[i % 2]:
            x0 = x0 + x1
            x1 = _rotl32(x1, r)
            x1 = x0 ^ x1
        x0 = x0 + ks[(i + 1) % 3]
        x1 = x1 + ks[(i + 2) % 3] + np.uint32(i + 1)
    return x0, x1


def _np_permutation(seed, n):
    key = (np.uint32(0), np.uint32(seed))
    x = np.arange(n, dtype=np.int32)
    num_rounds = int(np.ceil(3 * np.log(max(1, n)) / np.log(2**32 - 1)))
    for _ in range(num_rounds):
        b1, b2 = _threefry2x32_np(
            key[0], key[1], np.zeros(2, np.uint32), np.arange(2, dtype=np.uint32)
        )
        key, subkey = (b1[0], b2[0]), (b1[1], b2[1])
        b1, b2 = _threefry2x32_np(
            subkey[0], subkey[1], np.zeros(n, np.uint32),
            np.arange(n, dtype=np.uint32),
        )
        x = x[np.argsort(b1 ^ b2, kind="stable")]
    return x


_CHOICE = _np_permutation(42, N)[:S].astype(np.int32)
_WIDX = _CHOICE.reshape(WPB * NCH, CH)


def _sc_gather(pos_t, x, widx):
    mesh = plsc.VectorSubcoreMesh(core_axis_name="c", subcore_axis_name="s")

    @functools.partial(
        pl.kernel,
        mesh=mesh,
        compiler_params=pltpu.CompilerParams(needs_layout_passes=False),
        out_type=[
            jax.ShapeDtypeStruct((B, S, D), jnp.float32),
            jax.ShapeDtypeStruct((3, B, S), jnp.float32),
            jax.ShapeDtypeStruct((B, S), jnp.int32),
        ],
        scratch_types=[
            pltpu.VMEM((NCH, CH), jnp.int32),
            pltpu.VMEM((3, CH, D), jnp.float32),
            pltpu.VMEM((3 * N,), jnp.float32),
            pltpu.VMEM((3 * ROWS_PER_W,), jnp.float32),
            pltpu.SemaphoreType.DMA,
            pltpu.SemaphoreType.DMA,
            pltpu.SemaphoreType.DMA,
            pltpu.SemaphoreType.DMA,
            pltpu.SemaphoreType.DMA,
        ],
    )
    def k(pt, xf, gi, xout, pout, iout, idx_v, xbuf, ptbl, pbuf,
          sem0, sem1, sem2, wsem, psem):
        wid = lax.axis_index("s") * NC + lax.axis_index("c")
        b = wid // WPB
        q = wid % WPB
        rbase = q * ROWS_PER_W
        pltpu.sync_copy(gi.at[pl.ds(q * NCH, NCH)], idx_v)
        iws = [
            pltpu.async_copy(
                idx_v.at[c], iout.at[b].at[pl.ds(rbase + c * CH, CH)], psem
            )
            for c in range(NCH)
        ]

        pcps = [
            pltpu.async_copy(
                pt.at[c3, b], ptbl.at[pl.ds(c3 * N, N)], psem
            )
            for c3 in range(3)
        ]

        gsems = (sem0, sem1, sem2)
        xb = xf.at[b]
        xo = xout.at[b]
        cps = [
            pltpu.async_copy(xb.at[idx_v.at[c]], xbuf.at[c], gsems[c])
            for c in range(3)
        ]

        NV = ROWS_PER_W // L
        VPC = NV // NCH

        def pos_group(v):
            rows = idx_v[v // (CH // L), pl.ds((v % (CH // L)) * L, L)]
            for c3 in range(3):
                vals = plsc.load_gather(ptbl, [rows + c3 * N])
                pbuf[pl.ds(c3 * ROWS_PER_W + v * L, L)] = vals

        ws = [None] * NCH
        for c in range(NCH):
            p = c % 3
            cps[p].wait()
            ws[c] = pltpu.async_copy(
                xbuf.at[p], xo.at[pl.ds(rbase + c * CH, CH)], wsem
            )
            if c + 3 < NCH:
                ws[c].wait()
                cps[p] = pltpu.async_copy(
                    xb.at[idx_v.at[c + 3]], xbuf.at[p], gsems[p]
                )
            if c == 0:
                for cp in pcps:
                    cp.wait()
            for v in range(c * VPC, (c + 1) * VPC):
                pos_group(v)
        pws = [
            pltpu.async_copy(
                pbuf.at[pl.ds(c3 * ROWS_PER_W, ROWS_PER_W)],
                pout.at[c3, b].at[pl.ds(rbase, ROWS_PER_W)],
                psem,
            )
            for c3 in range(3)
        ]
        for c in range(NCH):
            if c + 3 >= NCH and ws[c] is not None:
                ws[c].wait()
        for iw in iws:
            iw.wait()
        for pw in pws:
            pw.wait()

    return k(pos_t, x, widx)


def kernel(pos, x):
    pos_t = jnp.transpose(pos, (2, 0, 1))
    xo, po_t, idx = _sc_gather(pos_t, x, jnp.asarray(_WIDX))
    return (idx, jnp.transpose(po_t, (1, 2, 0)), xo)
